# baseline (speedup 1.0000x reference)
import functools

import jax
import jax.numpy as jnp
from jax import lax
from jax.experimental import pallas as pl
from jax.experimental.pallas import tpu as pltpu
from jax.experimental.pallas import tpu_sc as plsc

N = 100000
D = 128
VOCAB_PAD = 84
CHUNK = 400
NCHUNK = N // CHUNK
NC = 2
NS = 16
NW = NC * NS
ITERS = (NCHUNK + NW - 1) // NW
IDXW = ITERS * CHUNK

_mesh = plsc.VectorSubcoreMesh(core_axis_name="c", subcore_axis_name="s")


@functools.partial(
    pl.kernel,
    mesh=_mesh,
    out_type=jax.ShapeDtypeStruct((N, D), jnp.float32),
    scratch_types=[
        pltpu.VMEM((IDXW,), jnp.int32),
        pltpu.VMEM((CHUNK, D), jnp.float32),
        pltpu.VMEM((CHUNK, D), jnp.float32),
        pltpu.VMEM_SHARED((VOCAB_PAD, D), jnp.float32),
        pltpu.SemaphoreType.DMA,
        pltpu.SemaphoreType.DMA,
        pltpu.SemaphoreType.DMA,
        pltpu.SemaphoreType.DMA,
        pltpu.SemaphoreType.DMA,
    ],
)
def _emb(table_hbm, idx_hbm, out_hbm,
         ibuf, rbuf0, rbuf1, table_sp, si, sg0, sg1, sw0, sw1):
    rbuf = (rbuf0, rbuf1)
    sg = (sg0, sg1)
    sw = (sw0, sw1)
    wid = lax.axis_index("s") * NC + lax.axis_index("c")

    load_base = jnp.minimum(wid * IDXW, N - IDXW)
    pltpu.async_copy(idx_hbm.at[pl.ds(load_base, IDXW)], ibuf, si)

    @pl.when(lax.axis_index("s") == 0)
    def _():
        pltpu.sync_copy(table_hbm, table_sp.at[pl.ds(1, VOCAB_PAD - 1)])

    plsc.subcore_barrier()
    pltpu.make_async_copy(idx_hbm.at[pl.ds(load_base, IDXW)], ibuf, si).wait()

    def step(i, carry):
        c = wid * ITERS + i

        def run(b):
            @pl.when(c < NCHUNK)
            def _():
                base = c * CHUNK

                @pl.when(i >= 2)
                def _():
                    pltpu.make_async_copy(
                        rbuf[b], out_hbm.at[pl.ds(base, CHUNK)], sw[b]).wait()

                local = base - load_base
                pltpu.async_copy(
                    table_sp.at[ibuf.at[pl.ds(local, CHUNK)]], rbuf[b],
                    sg[b]).wait()
                pltpu.async_copy(rbuf[b], out_hbm.at[pl.ds(base, CHUNK)], sw[b])

        @pl.when(i % 2 == 0)
        def _():
            run(0)

        @pl.when(i % 2 == 1)
        def _():
            run(1)

        return carry

    lax.fori_loop(0, ITERS, step, 0)

    for i in range(max(ITERS - 2, 0), ITERS):
        b = i & 1
        c = wid * ITERS + i

        @pl.when(c < NCHUNK)
        def _(b=b, c=c):
            pltpu.make_async_copy(
                rbuf[b], out_hbm.at[pl.ds(c * CHUNK, CHUNK)], sw[b]).wait()


def kernel(Z, W):
    return _emb(W, Z.astype(jnp.int32))


# speedup vs baseline: 1.0098x; 1.0098x over previous
import functools

import jax
import jax.numpy as jnp
from jax import lax
from jax.experimental import pallas as pl
from jax.experimental.pallas import tpu as pltpu
from jax.experimental.pallas import tpu_sc as plsc

N = 100000
D = 128
VOCAB_PAD = 84
CHUNK = 400
NCHUNK = N // CHUNK
NC = 2
NS = 16
NW = NC * NS
ITERS = (NCHUNK + NW - 1) // NW

_mesh = plsc.VectorSubcoreMesh(core_axis_name="c", subcore_axis_name="s")


@functools.partial(
    pl.kernel,
    mesh=_mesh,
    out_type=jax.ShapeDtypeStruct((N, D), jnp.float32),
    scratch_types=[
        pltpu.VMEM((CHUNK,), jnp.int32),
        pltpu.VMEM((CHUNK,), jnp.int32),
        pltpu.VMEM((CHUNK, D), jnp.float32),
        pltpu.VMEM((CHUNK, D), jnp.float32),
        pltpu.VMEM_SHARED((VOCAB_PAD, D), jnp.float32),
        pltpu.SemaphoreType.DMA,
        pltpu.SemaphoreType.DMA,
        pltpu.SemaphoreType.DMA,
        pltpu.SemaphoreType.DMA,
        pltpu.SemaphoreType.DMA,
        pltpu.SemaphoreType.DMA,
    ],
)
def _emb(table_hbm, idx_hbm, out_hbm,
         ibuf0, ibuf1, rbuf0, rbuf1, table_sp, si0, si1, sg0, sg1, sw0, sw1):
    ibuf = (ibuf0, ibuf1)
    rbuf = (rbuf0, rbuf1)
    si = (si0, si1)
    sg = (sg0, sg1)
    sw = (sw0, sw1)
    wid = lax.axis_index("s") * NC + lax.axis_index("c")

    for i in range(2):
        c = wid + i * NW

        @pl.when(c < NCHUNK)
        def _(i=i, c=c):
            pltpu.async_copy(idx_hbm.at[pl.ds(c * CHUNK, CHUNK)], ibuf[i], si[i])

    @pl.when(lax.axis_index("s") == 0)
    def _():
        pltpu.sync_copy(table_hbm, table_sp.at[pl.ds(1, VOCAB_PAD - 1)])

    plsc.subcore_barrier()

    def step(i, carry):
        c = wid + i * NW

        def run(b):
            @pl.when(c < NCHUNK)
            def _():
                base = c * CHUNK
                pltpu.make_async_copy(
                    idx_hbm.at[pl.ds(base, CHUNK)], ibuf[b], si[b]).wait()

                @pl.when(i >= 2)
                def _():
                    pltpu.make_async_copy(
                        rbuf[b], out_hbm.at[pl.ds(base, CHUNK)], sw[b]).wait()

                pltpu.async_copy(table_sp.at[ibuf[b]], rbuf[b], sg[b]).wait()
                pltpu.async_copy(rbuf[b], out_hbm.at[pl.ds(base, CHUNK)], sw[b])

            c2 = wid + (i + 2) * NW

            @pl.when(c2 < NCHUNK)
            def _():
                pltpu.async_copy(
                    idx_hbm.at[pl.ds(c2 * CHUNK, CHUNK)], ibuf[b], si[b])

        @pl.when(i % 2 == 0)
        def _():
            run(0)

        @pl.when(i % 2 == 1)
        def _():
            run(1)

        return carry

    lax.fori_loop(0, ITERS, step, 0)

    for i in range(max(ITERS - 2, 0), ITERS):
        b = i & 1
        c = wid + i * NW

        @pl.when(c < NCHUNK)
        def _(b=b, c=c):
            pltpu.make_async_copy(
                rbuf[b], out_hbm.at[pl.ds(c * CHUNK, CHUNK)], sw[b]).wait()


def kernel(Z, W):
    return _emb(W, Z.astype(jnp.int32))


# speedup vs baseline: 1.0127x; 1.0029x over previous
import functools

import jax
import jax.numpy as jnp
from jax import lax
from jax.experimental import pallas as pl
from jax.experimental.pallas import tpu as pltpu
from jax.experimental.pallas import tpu_sc as plsc

N = 100000
D = 128
VOCAB_PAD = 84
CHUNK = 400
NCHUNK = N // CHUNK
NC = 2
NS = 16
NW = NC * NS
ITERS = (NCHUNK + NW - 1) // NW

_mesh = plsc.VectorSubcoreMesh(core_axis_name="c", subcore_axis_name="s")


@functools.partial(
    pl.kernel,
    mesh=_mesh,
    out_type=jax.ShapeDtypeStruct((N, D), jnp.float32),
    scratch_types=[
        pltpu.VMEM((CHUNK,), jnp.int32),
        pltpu.VMEM((CHUNK,), jnp.int32),
        pltpu.VMEM((CHUNK, D), jnp.float32),
        pltpu.VMEM((CHUNK, D), jnp.float32),
        pltpu.VMEM_SHARED((VOCAB_PAD, D), jnp.float32),
        pltpu.SemaphoreType.DMA,
        pltpu.SemaphoreType.DMA,
        pltpu.SemaphoreType.DMA,
        pltpu.SemaphoreType.DMA,
        pltpu.SemaphoreType.DMA,
        pltpu.SemaphoreType.DMA,
    ],
)
def _emb(table_hbm, idx_hbm, out_hbm,
         ibuf0, ibuf1, rbuf0, rbuf1, table_sp, si0, si1, sg0, sg1, sw0, sw1):
    ibuf = (ibuf0, ibuf1)
    rbuf = (rbuf0, rbuf1)
    si = (si0, si1)
    sg = (sg0, sg1)
    sw = (sw0, sw1)
    wid = lax.axis_index("s") * NC + lax.axis_index("c")

    for i in range(2):
        c = wid + i * NW

        @pl.when(c < NCHUNK)
        def _(i=i, c=c):
            pltpu.async_copy(idx_hbm.at[pl.ds(c * CHUNK, CHUNK)], ibuf[i], si[i])

    sid = lax.axis_index("s")

    @pl.when(sid < 10)
    def _():
        off = pl.multiple_of(8 * sid, 8)
        pltpu.sync_copy(table_hbm.at[pl.ds(off, 8)],
                        table_sp.at[pl.ds(1 + off, 8)])

    @pl.when(sid == 10)
    def _():
        pltpu.sync_copy(table_hbm.at[pl.ds(80, 3)],
                        table_sp.at[pl.ds(81, 3)])

    plsc.subcore_barrier()

    def step(i, carry):
        c = wid + i * NW

        def run(b):
            @pl.when(c < NCHUNK)
            def _():
                base = c * CHUNK
                pltpu.make_async_copy(
                    idx_hbm.at[pl.ds(base, CHUNK)], ibuf[b], si[b]).wait()

                @pl.when(i >= 2)
                def _():
                    pltpu.make_async_copy(
                        rbuf[b], out_hbm.at[pl.ds(base, CHUNK)], sw[b]).wait()

                pltpu.async_copy(table_sp.at[ibuf[b]], rbuf[b], sg[b]).wait()
                pltpu.async_copy(rbuf[b], out_hbm.at[pl.ds(base, CHUNK)], sw[b])

            c2 = wid + (i + 2) * NW

            @pl.when(c2 < NCHUNK)
            def _():
                pltpu.async_copy(
                    idx_hbm.at[pl.ds(c2 * CHUNK, CHUNK)], ibuf[b], si[b])

        @pl.when(i % 2 == 0)
        def _():
            run(0)

        @pl.when(i % 2 == 1)
        def _():
            run(1)

        return carry

    lax.fori_loop(0, ITERS, step, 0)

    for i in range(max(ITERS - 2, 0), ITERS):
        b = i & 1
        c = wid + i * NW

        @pl.when(c < NCHUNK)
        def _(b=b, c=c):
            pltpu.make_async_copy(
                rbuf[b], out_hbm.at[pl.ds(c * CHUNK, CHUNK)], sw[b]).wait()


def kernel(Z, W):
    return _emb(W, Z.astype(jnp.int32))


# speedup vs baseline: 1.0143x; 1.0015x over previous
import functools

import jax
import jax.numpy as jnp
from jax import lax
from jax.experimental import pallas as pl
from jax.experimental.pallas import tpu as pltpu
from jax.experimental.pallas import tpu_sc as plsc

N = 100000
D = 128
VOCAB_PAD = 84
CHUNK = 400
NCHUNK = N // CHUNK
NC = 2
NS = 16
NW = NC * NS
ITERS = (NCHUNK + NW - 1) // NW

_mesh = plsc.VectorSubcoreMesh(core_axis_name="c", subcore_axis_name="s")


@functools.partial(
    pl.kernel,
    mesh=_mesh,
    out_type=jax.ShapeDtypeStruct((N, D), jnp.float32),
    scratch_types=[
        pltpu.VMEM((CHUNK,), jnp.int32),
        pltpu.VMEM((CHUNK,), jnp.int32),
        pltpu.VMEM((CHUNK, D), jnp.float32),
        pltpu.VMEM((CHUNK, D), jnp.float32),
        pltpu.VMEM_SHARED((VOCAB_PAD, D), jnp.float32),
        pltpu.SemaphoreType.DMA,
        pltpu.SemaphoreType.DMA,
        pltpu.SemaphoreType.DMA,
        pltpu.SemaphoreType.DMA,
        pltpu.SemaphoreType.DMA,
        pltpu.SemaphoreType.DMA,
    ],
)
def _emb(table_hbm, idx_hbm, out_hbm,
         ibuf0, ibuf1, rbuf0, rbuf1, table_sp, si0, si1, sg0, sg1, sw0, sw1):
    ibuf = (ibuf0, ibuf1)
    rbuf = (rbuf0, rbuf1)
    si = (si0, si1)
    sg = (sg0, sg1)
    sw = (sw0, sw1)
    wid = lax.axis_index("s") * NC + lax.axis_index("c")

    for i in range(2):
        c = wid + i * NW

        @pl.when(c < NCHUNK)
        def _(i=i, c=c):
            pltpu.async_copy(idx_hbm.at[pl.ds(c * CHUNK, CHUNK)], ibuf[i], si[i])

    sid = lax.axis_index("s")

    @pl.when(sid < 10)
    def _():
        off = pl.multiple_of(8 * sid, 8)
        pltpu.sync_copy(table_hbm.at[pl.ds(off, 8)],
                        table_sp.at[pl.ds(1 + off, 8)])

    @pl.when(sid == 10)
    def _():
        pltpu.sync_copy(table_hbm.at[pl.ds(80, 3)],
                        table_sp.at[pl.ds(81, 3)])

    plsc.subcore_barrier()

    def step(i, carry):
        c = wid + i * NW

        def run(b):
            @pl.when(c < NCHUNK)
            def _():
                base = c * CHUNK
                pltpu.make_async_copy(
                    idx_hbm.at[pl.ds(base, CHUNK)], ibuf[b], si[b]).wait()

                @pl.when(i >= 2)
                def _():
                    pltpu.make_async_copy(
                        rbuf[b], out_hbm.at[pl.ds(base, CHUNK)], sw[b]).wait()

                H = CHUNK // 2
                for h in range(2):
                    pltpu.async_copy(
                        table_sp.at[ibuf[b].at[pl.ds(h * H, H)]],
                        rbuf[b].at[pl.ds(h * H, H)], sg[b]).wait()
                    pltpu.async_copy(rbuf[b].at[pl.ds(h * H, H)],
                                     out_hbm.at[pl.ds(base + h * H, H)], sw[b])

            c2 = wid + (i + 2) * NW

            @pl.when(c2 < NCHUNK)
            def _():
                pltpu.async_copy(
                    idx_hbm.at[pl.ds(c2 * CHUNK, CHUNK)], ibuf[b], si[b])

        @pl.when(i % 2 == 0)
        def _():
            run(0)

        @pl.when(i % 2 == 1)
        def _():
            run(1)

        return carry

    lax.fori_loop(0, ITERS, step, 0)

    for i in range(max(ITERS - 2, 0), ITERS):
        b = i & 1
        c = wid + i * NW

        @pl.when(c < NCHUNK)
        def _(b=b, c=c):
            pltpu.make_async_copy(
                rbuf[b], out_hbm.at[pl.ds(c * CHUNK, CHUNK)], sw[b]).wait()


def kernel(Z, W):
    return _emb(W, Z.astype(jnp.int32))
